# Initial kernel scaffold; baseline (speedup 1.0000x reference)
import jax
import jax.numpy as jnp
from jax.experimental import pallas as pl


def kernel(x, edge_index, edge_attr, batch, emb, e1_W, e1_b, e2_W, e2_b, n1_W, n1_b, n2_W, n2_b, lin_W, lin_b):
    raise NotImplementedError("write your pallas kernel here")


# speedup vs baseline: 1.9828x; 1.9828x over previous
import functools

import jax
import jax.numpy as jnp
from jax import lax
from jax.experimental import pallas as pl
from jax.experimental.pallas import tpu as pltpu
from jax.experimental.pallas import tpu_sc as plsc

N = 10000
E = 320000
H = 128
RBF = 128
L = 3
G = 64

NC = 2
NS = 16
NW = NC * NS

CHI = 80
KJ = 8
CH = CHI * KJ

BN = 1000
BE = 2000
NEB = E // BE

F32 = jnp.float32
HI = lax.Precision.HIGHEST

def _mesh():
    return plsc.VectorSubcoreMesh(core_axis_name="c", subcore_axis_name="s")


def _sc_gather(table, idx2d):
    M = idx2d.shape[0] * CHI
    n_ch = M // CH

    @functools.partial(
        pl.kernel,
        mesh=_mesh(),
        out_type=jax.ShapeDtypeStruct((M, H), F32),
        scratch_types=[
            pltpu.VMEM((KJ, CHI), jnp.int32),
            pltpu.VMEM((CH, H), F32),
            pltpu.SemaphoreType.DMA,
        ],
    )
    def k(table_hbm, idx_hbm, out_hbm, idx_v, rows_v, sem):
        wid = lax.axis_index("s") * NC + lax.axis_index("c")

        @pl.loop(wid, n_ch, step=NW)
        def _(cc):
            pltpu.sync_copy(idx_hbm.at[pl.ds(cc * KJ, KJ)], idx_v)
            copies = [
                pltpu.async_copy(
                    table_hbm.at[idx_v.at[j]],
                    rows_v.at[pl.ds(j * CHI, CHI)],
                    sem,
                )
                for j in range(KJ)
            ]
            for c in copies:
                c.wait()
            pltpu.sync_copy(rows_v, out_hbm.at[pl.ds(cc * CH, CH)])

    return k(table, idx2d)


NHALF = N // NC
ACC_R = NHALF + 8


def _sc_scatter_add(m, ridx3d, zeros_acc):
    n_ch = E // CH
    ZB = 8

    @functools.partial(
        pl.kernel,
        mesh=_mesh(),
        out_type=jax.ShapeDtypeStruct((NC, NHALF, H), F32),
        scratch_types=[
            pltpu.VMEM((KJ, CHI), jnp.int32),
            pltpu.VMEM((CH, H), F32),
            pltpu.VMEM_SHARED((ACC_R, H), F32),
            pltpu.SemaphoreType.DMA,
        ],
    )
    def k(m_hbm, idx_hbm, zero_hbm, out_hbm, idx_v, m_v, acc, sem):
        cid = lax.axis_index("c")
        sid = lax.axis_index("s")

        @pl.loop(sid, ACC_R // ZB, step=NS)
        def _(z):
            pltpu.sync_copy(
                zero_hbm.at[pl.ds(z * ZB, ZB)], acc.at[pl.ds(z * ZB, ZB)]
            )

        plsc.subcore_barrier()

        @pl.loop(sid, n_ch, step=NS)
        def _(cc):
            pltpu.sync_copy(idx_hbm.at[cid, pl.ds(cc * KJ, KJ)], idx_v)
            pltpu.sync_copy(m_hbm.at[pl.ds(cc * CH, CH)], m_v)
            for j in range(KJ):
                pltpu.sync_copy(
                    m_v.at[pl.ds(j * CHI, CHI)],
                    acc.at[idx_v.at[j]],
                    add=True,
                )

        plsc.subcore_barrier()

        DB = 40
        @pl.loop(sid, NHALF // DB, step=NS)
        def _(z):
            pltpu.sync_copy(
                acc.at[pl.ds(z * DB, DB)],
                out_hbm.at[cid, pl.ds(z * DB, DB)],
            )

    return k(m, ridx3d, zeros_acc)


def _silu(v):
    return v * jax.nn.sigmoid(v)


def _tc_init(x3, embp, wa, wb):
    def body(x_ref, emb_ref, wa_ref, wb_ref, h0_ref, t_ref):
        xv = x_ref[0, 0, :]
        io = lax.broadcasted_iota(jnp.int32, (BN, 128), 1)
        oh = (xv[:, None] == io).astype(F32)
        h0 = jnp.dot(oh, emb_ref[...], precision=HI)
        h0_ref[...] = h0
        t_ref[0] = jnp.dot(h0, wa_ref[...], precision=HI)
        t_ref[1] = jnp.dot(h0, wb_ref[...], precision=HI)

    return pl.pallas_call(
        body,
        grid=(N // BN,),
        in_specs=[
            pl.BlockSpec((1, 1, BN), lambda i: (i, 0, 0)),
            pl.BlockSpec((128, H), lambda i: (0, 0)),
            pl.BlockSpec((H, H), lambda i: (0, 0)),
            pl.BlockSpec((H, H), lambda i: (0, 0)),
        ],
        out_specs=[
            pl.BlockSpec((BN, H), lambda i: (i, 0)),
            pl.BlockSpec((2, BN, H), lambda i: (0, i, 0)),
        ],
        out_shape=[
            jax.ShapeDtypeStruct((N, H), F32),
            jax.ShapeDtypeStruct((2, N, H), F32),
        ],
    )(x3, embp, wa, wb)


def _tc_edge(gfull, edge_attr, wc, e2w, b1, b2):
    def body(ga_ref, gb_ref, ea_ref, wc_ref, e2_ref, b1_ref, b2_ref, m_ref):
        t = (
            ga_ref[...]
            + gb_ref[...]
            + jnp.dot(ea_ref[...], wc_ref[...], precision=HI)
            + b1_ref[...]
        )
        t = _silu(t)
        m = jnp.dot(t, e2_ref[...], precision=HI) + b2_ref[...]
        m_ref[...] = _silu(m)

    return pl.pallas_call(
        body,
        grid=(NEB,),
        in_specs=[
            pl.BlockSpec((BE, H), lambda i: (i, 0)),
            pl.BlockSpec((BE, H), lambda i: (i + NEB, 0)),
            pl.BlockSpec((BE, H), lambda i: (i, 0)),
            pl.BlockSpec((H, H), lambda i: (0, 0)),
            pl.BlockSpec((H, H), lambda i: (0, 0)),
            pl.BlockSpec((1, H), lambda i: (0, 0)),
            pl.BlockSpec((1, H), lambda i: (0, 0)),
        ],
        out_specs=pl.BlockSpec((BE, H), lambda i: (i, 0)),
        out_shape=jax.ShapeDtypeStruct((E, H), F32),
    )(gfull, gfull, edge_attr, wc, e2w, b1, b2)


def _tc_node(h, agg_nh, n1a, n1b, nb1, n2w, nb2, wa=None, wb=None):
    with_t = wa is not None

    def body(h_ref, agg_ref, a1, a2, b1r, w2, b2r, *rest):
        if with_t:
            wa_ref, wb_ref, hn_ref, t_ref = rest
        else:
            (hn_ref,) = rest
        h_v = h_ref[...]
        agg = agg_ref[...]
        u = (
            jnp.dot(h_v, a1[...], precision=HI)
            + jnp.dot(agg, a2[...], precision=HI)
            + b1r[...]
        )
        u = _silu(u)
        hn = h_v + jnp.dot(u, w2[...], precision=HI) + b2r[...]
        hn_ref[...] = hn
        if with_t:
            t_ref[0] = jnp.dot(hn, wa_ref[...], precision=HI)
            t_ref[1] = jnp.dot(hn, wb_ref[...], precision=HI)

    in_specs = [
        pl.BlockSpec((BN, H), lambda i: (i, 0)),
        pl.BlockSpec((BN, H), lambda i: (i, 0)),
        pl.BlockSpec((H, H), lambda i: (0, 0)),
        pl.BlockSpec((H, H), lambda i: (0, 0)),
        pl.BlockSpec((1, H), lambda i: (0, 0)),
        pl.BlockSpec((H, H), lambda i: (0, 0)),
        pl.BlockSpec((1, H), lambda i: (0, 0)),
    ]
    args = [h, agg_nh, n1a, n1b, nb1, n2w, nb2]
    out_specs = [pl.BlockSpec((BN, H), lambda i: (i, 0))]
    out_shape = [jax.ShapeDtypeStruct((N, H), F32)]
    if with_t:
        in_specs += [
            pl.BlockSpec((H, H), lambda i: (0, 0)),
            pl.BlockSpec((H, H), lambda i: (0, 0)),
        ]
        args += [wa, wb]
        out_specs.append(pl.BlockSpec((2, BN, H), lambda i: (0, i, 0)))
        out_shape.append(jax.ShapeDtypeStruct((2, N, H), F32))

    res = pl.pallas_call(
        body,
        grid=(N // BN,),
        in_specs=in_specs,
        out_specs=out_specs,
        out_shape=out_shape,
    )(*args)
    return res if with_t else (res[0], None)


def _tc_pool(batch2, h, linw, linb):
    def body(b_ref, h_ref, w_ref, bias_ref, o_ref):
        bv = b_ref[0, :]
        io = lax.broadcasted_iota(jnp.int32, (N, G), 1)
        oh = (bv[:, None] == io).astype(F32)
        h_v = h_ref[...]
        sums = lax.dot_general(
            oh, h_v, (((0,), (0,)), ((), ())), precision=HI
        )
        ones = jnp.ones((N, 1), F32)
        counts = lax.dot_general(
            oh, ones, (((0,), (0,)), ((), ())), precision=HI
        )
        pooled = sums / jnp.maximum(counts, 1.0)
        o_ref[...] = (
            jnp.dot(jnp.maximum(pooled, 0.0), w_ref[...], precision=HI)
            + bias_ref[...]
        )

    return pl.pallas_call(
        body,
        grid=(1,),
        in_specs=[
            pl.BlockSpec((1, N), lambda i: (0, 0)),
            pl.BlockSpec((N, H), lambda i: (0, 0)),
            pl.BlockSpec((H, H), lambda i: (0, 0)),
            pl.BlockSpec((1, H), lambda i: (0, 0)),
        ],
        out_specs=pl.BlockSpec((G, H), lambda i: (0, 0)),
        out_shape=jax.ShapeDtypeStruct((G, H), F32),
    )(batch2, h, linw, linb)


def kernel(x, edge_index, edge_attr, batch, emb, e1_W, e1_b, e2_W, e2_b,
           n1_W, n1_b, n2_W, n2_b, lin_W, lin_b):
    x = x.astype(jnp.int32)
    row = edge_index[0]
    col = edge_index[1]
    idx2 = jnp.concatenate([row, col + N]).reshape(2 * E // CHI, CHI)
    dummy = NHALF + (row & 7)
    ridx = jnp.stack([
        jnp.where(row < NHALF, row, dummy),
        jnp.where(row >= NHALF, row - NHALF, dummy),
    ]).reshape(NC, E // CHI, CHI)
    zeros_acc = jnp.zeros((ACC_R, H), F32)
    embp = jnp.pad(emb, ((0, 128 - emb.shape[0]), (0, 0)))

    wa = [e1_W[l, 0:H] for l in range(L)]
    wb = [e1_W[l, H:2 * H] for l in range(L)]
    wc = [e1_W[l, 2 * H:] for l in range(L)]
    n1a = [n1_W[l, 0:H] for l in range(L)]
    n1b = [n1_W[l, H:] for l in range(L)]

    h, t = _tc_init(x.reshape(N // BN, 1, BN), embp, wa[0], wb[0])
    for l in range(L):
        gfull = _sc_gather(t.reshape(2 * N, H), idx2)
        m = _tc_edge(
            gfull, edge_attr, wc[l], e2_W[l],
            e1_b[l].reshape(1, H), e2_b[l].reshape(1, H),
        )
        parts = _sc_scatter_add(m, ridx, zeros_acc)
        last = l == L - 1
        h, t = _tc_node(
            h, parts.reshape(N, H), n1a[l], n1b[l], n1_b[l].reshape(1, H),
            n2_W[l], n2_b[l].reshape(1, H),
            None if last else wa[l + 1], None if last else wb[l + 1],
        )

    return _tc_pool(batch.reshape(1, N), h, lin_W, lin_b.reshape(1, H))


# speedup vs baseline: 2.5918x; 1.3071x over previous
import functools

import jax
import jax.numpy as jnp
from jax import lax
from jax.experimental import pallas as pl
from jax.experimental.pallas import tpu as pltpu
from jax.experimental.pallas import tpu_sc as plsc

N = 10000
E = 320000
H = 128
RBF = 128
L = 3
G = 64

NC = 2
NS = 16
NW = NC * NS

CHI = 80
KJ = 8
CH = CHI * KJ

BN = 1000
BE = 2000
NEB = E // BE

F32 = jnp.float32
HI = lax.Precision.HIGHEST

def _mesh():
    return plsc.VectorSubcoreMesh(core_axis_name="c", subcore_axis_name="s")


def _sc_gather(table, idx2d):
    M = idx2d.shape[0] * CHI
    n_ch = M // CH

    @functools.partial(
        pl.kernel,
        mesh=_mesh(),
        out_type=jax.ShapeDtypeStruct((M, H), F32),
        scratch_types=[
            pltpu.VMEM((KJ, CHI), jnp.int32),
            pltpu.VMEM((CH, H), F32),
            pltpu.SemaphoreType.DMA,
        ],
    )
    def k(table_hbm, idx_hbm, out_hbm, idx_v, rows_v, sem):
        wid = lax.axis_index("s") * NC + lax.axis_index("c")

        @pl.loop(wid, n_ch, step=NW)
        def _(cc):
            pltpu.sync_copy(idx_hbm.at[pl.ds(cc * KJ, KJ)], idx_v)
            copies = [
                pltpu.async_copy(
                    table_hbm.at[idx_v.at[j]],
                    rows_v.at[pl.ds(j * CHI, CHI)],
                    sem,
                )
                for j in range(KJ)
            ]
            for c in copies:
                c.wait()
            pltpu.sync_copy(rows_v, out_hbm.at[pl.ds(cc * CH, CH)])

    return k(table, idx2d)


NHALF = N // NC
ACC_R = NHALF + 8


def _sc_scatter_add(m, ridx3d, zeros_acc):
    n_ch = E // CH
    ZB = 8

    @functools.partial(
        pl.kernel,
        mesh=_mesh(),
        out_type=jax.ShapeDtypeStruct((NC, NHALF, H), F32),
        scratch_types=[
            pltpu.VMEM((KJ, CHI), jnp.int32),
            pltpu.VMEM((CH, H), F32),
            pltpu.VMEM_SHARED((ACC_R, H), F32),
            pltpu.SemaphoreType.DMA,
        ],
    )
    def k(m_hbm, idx_hbm, zero_hbm, out_hbm, idx_v, m_v, acc, sem):
        cid = lax.axis_index("c")
        sid = lax.axis_index("s")

        @pl.loop(sid, ACC_R // ZB, step=NS)
        def _(z):
            pltpu.sync_copy(
                zero_hbm.at[pl.ds(z * ZB, ZB)], acc.at[pl.ds(z * ZB, ZB)]
            )

        plsc.subcore_barrier()

        @pl.loop(sid, n_ch, step=NS)
        def _(cc):
            pltpu.sync_copy(idx_hbm.at[cid, pl.ds(cc * KJ, KJ)], idx_v)
            pltpu.sync_copy(m_hbm.at[pl.ds(cc * CH, CH)], m_v)
            for j in range(KJ):
                pltpu.sync_copy(
                    m_v.at[pl.ds(j * CHI, CHI)],
                    acc.at[idx_v.at[j]],
                    add=True,
                )

        plsc.subcore_barrier()

        DB = 40
        @pl.loop(sid, NHALF // DB, step=NS)
        def _(z):
            pltpu.sync_copy(
                acc.at[pl.ds(z * DB, DB)],
                out_hbm.at[cid, pl.ds(z * DB, DB)],
            )

    return k(m, ridx3d, zeros_acc)


def _silu(v):
    return v * jax.nn.sigmoid(v)


def _tc_init(x3, embp, wa, wb):
    def body(x_ref, emb_ref, wa_ref, wb_ref, h0_ref, t_ref):
        xv = x_ref[0, 0, :]
        io = lax.broadcasted_iota(jnp.int32, (BN, 128), 1)
        oh = (xv[:, None] == io).astype(F32)
        h0 = jnp.dot(oh, emb_ref[...], precision=HI)
        h0_ref[...] = h0
        t_ref[0] = jnp.dot(h0, wa_ref[...], precision=HI)
        t_ref[1] = jnp.dot(h0, wb_ref[...], precision=HI)

    return pl.pallas_call(
        body,
        grid=(N // BN,),
        in_specs=[
            pl.BlockSpec((1, 1, BN), lambda i: (i, 0, 0)),
            pl.BlockSpec((128, H), lambda i: (0, 0)),
            pl.BlockSpec((H, H), lambda i: (0, 0)),
            pl.BlockSpec((H, H), lambda i: (0, 0)),
        ],
        out_specs=[
            pl.BlockSpec((BN, H), lambda i: (i, 0)),
            pl.BlockSpec((2, BN, H), lambda i: (0, i, 0)),
        ],
        out_shape=[
            jax.ShapeDtypeStruct((N, H), F32),
            jax.ShapeDtypeStruct((2, N, H), F32),
        ],
    )(x3, embp, wa, wb)


def _tc_edge(gfull, edge_attr, wc, e2w, b1, b2):
    def body(ga_ref, gb_ref, ea_ref, wc_ref, e2_ref, b1_ref, b2_ref, m_ref):
        t = (
            ga_ref[...]
            + gb_ref[...]
            + jnp.dot(ea_ref[...], wc_ref[...],
                      preferred_element_type=F32)
            + b1_ref[...]
        )
        t = _silu(t)
        m = (
            jnp.dot(t.astype(jnp.bfloat16), e2_ref[...],
                    preferred_element_type=F32)
            + b2_ref[...]
        )
        m_ref[...] = _silu(m)

    return pl.pallas_call(
        body,
        grid=(NEB,),
        in_specs=[
            pl.BlockSpec((BE, H), lambda i: (i, 0)),
            pl.BlockSpec((BE, H), lambda i: (i + NEB, 0)),
            pl.BlockSpec((BE, H), lambda i: (i, 0)),
            pl.BlockSpec((H, H), lambda i: (0, 0)),
            pl.BlockSpec((H, H), lambda i: (0, 0)),
            pl.BlockSpec((1, H), lambda i: (0, 0)),
            pl.BlockSpec((1, H), lambda i: (0, 0)),
        ],
        out_specs=pl.BlockSpec((BE, H), lambda i: (i, 0)),
        out_shape=jax.ShapeDtypeStruct((E, H), F32),
    )(gfull, gfull, edge_attr, wc.astype(jnp.bfloat16),
      e2w.astype(jnp.bfloat16), b1, b2)


def _tc_node(h, agg_nh, n1a, n1b, nb1, n2w, nb2, wa=None, wb=None):
    with_t = wa is not None

    def body(h_ref, agg_ref, a1, a2, b1r, w2, b2r, *rest):
        if with_t:
            wa_ref, wb_ref, hn_ref, t_ref = rest
        else:
            (hn_ref,) = rest
        h_v = h_ref[...]
        agg = agg_ref[...]
        u = (
            jnp.dot(h_v, a1[...], precision=HI)
            + jnp.dot(agg, a2[...], precision=HI)
            + b1r[...]
        )
        u = _silu(u)
        hn = h_v + jnp.dot(u, w2[...], precision=HI) + b2r[...]
        hn_ref[...] = hn
        if with_t:
            t_ref[0] = jnp.dot(hn, wa_ref[...], precision=HI)
            t_ref[1] = jnp.dot(hn, wb_ref[...], precision=HI)

    in_specs = [
        pl.BlockSpec((BN, H), lambda i: (i, 0)),
        pl.BlockSpec((BN, H), lambda i: (i, 0)),
        pl.BlockSpec((H, H), lambda i: (0, 0)),
        pl.BlockSpec((H, H), lambda i: (0, 0)),
        pl.BlockSpec((1, H), lambda i: (0, 0)),
        pl.BlockSpec((H, H), lambda i: (0, 0)),
        pl.BlockSpec((1, H), lambda i: (0, 0)),
    ]
    args = [h, agg_nh, n1a, n1b, nb1, n2w, nb2]
    out_specs = [pl.BlockSpec((BN, H), lambda i: (i, 0))]
    out_shape = [jax.ShapeDtypeStruct((N, H), F32)]
    if with_t:
        in_specs += [
            pl.BlockSpec((H, H), lambda i: (0, 0)),
            pl.BlockSpec((H, H), lambda i: (0, 0)),
        ]
        args += [wa, wb]
        out_specs.append(pl.BlockSpec((2, BN, H), lambda i: (0, i, 0)))
        out_shape.append(jax.ShapeDtypeStruct((2, N, H), F32))

    res = pl.pallas_call(
        body,
        grid=(N // BN,),
        in_specs=in_specs,
        out_specs=out_specs,
        out_shape=out_shape,
    )(*args)
    return res if with_t else (res[0], None)


def _tc_pool(batch2, h, linw, linb):
    def body(b_ref, h_ref, w_ref, bias_ref, o_ref):
        bv = b_ref[0, :]
        io = lax.broadcasted_iota(jnp.int32, (N, G), 1)
        oh = (bv[:, None] == io).astype(F32)
        h_v = h_ref[...]
        sums = lax.dot_general(
            oh, h_v, (((0,), (0,)), ((), ())), precision=HI
        )
        ones = jnp.ones((N, 1), F32)
        counts = lax.dot_general(
            oh, ones, (((0,), (0,)), ((), ())), precision=HI
        )
        pooled = sums / jnp.maximum(counts, 1.0)
        o_ref[...] = (
            jnp.dot(jnp.maximum(pooled, 0.0), w_ref[...], precision=HI)
            + bias_ref[...]
        )

    return pl.pallas_call(
        body,
        grid=(1,),
        in_specs=[
            pl.BlockSpec((1, N), lambda i: (0, 0)),
            pl.BlockSpec((N, H), lambda i: (0, 0)),
            pl.BlockSpec((H, H), lambda i: (0, 0)),
            pl.BlockSpec((1, H), lambda i: (0, 0)),
        ],
        out_specs=pl.BlockSpec((G, H), lambda i: (0, 0)),
        out_shape=jax.ShapeDtypeStruct((G, H), F32),
    )(batch2, h, linw, linb)


def kernel(x, edge_index, edge_attr, batch, emb, e1_W, e1_b, e2_W, e2_b,
           n1_W, n1_b, n2_W, n2_b, lin_W, lin_b):
    x = x.astype(jnp.int32)
    row = edge_index[0]
    col = edge_index[1]
    idx2 = jnp.concatenate([row, col + N]).reshape(2 * E // CHI, CHI)
    dummy = NHALF + (row & 7)
    ridx = jnp.stack([
        jnp.where(row < NHALF, row, dummy),
        jnp.where(row >= NHALF, row - NHALF, dummy),
    ]).reshape(NC, E // CHI, CHI)
    zeros_acc = jnp.zeros((ACC_R, H), F32)
    ea16 = edge_attr.astype(jnp.bfloat16)
    embp = jnp.pad(emb, ((0, 128 - emb.shape[0]), (0, 0)))

    wa = [e1_W[l, 0:H] for l in range(L)]
    wb = [e1_W[l, H:2 * H] for l in range(L)]
    wc = [e1_W[l, 2 * H:] for l in range(L)]
    n1a = [n1_W[l, 0:H] for l in range(L)]
    n1b = [n1_W[l, H:] for l in range(L)]

    h, t = _tc_init(x.reshape(N // BN, 1, BN), embp, wa[0], wb[0])
    for l in range(L):
        gfull = _sc_gather(t.reshape(2 * N, H), idx2)
        m = _tc_edge(
            gfull, ea16, wc[l], e2_W[l],
            e1_b[l].reshape(1, H), e2_b[l].reshape(1, H),
        )
        parts = _sc_scatter_add(m, ridx, zeros_acc)
        last = l == L - 1
        h, t = _tc_node(
            h, parts.reshape(N, H), n1a[l], n1b[l], n1_b[l].reshape(1, H),
            n2_W[l], n2_b[l].reshape(1, H),
            None if last else wa[l + 1], None if last else wb[l + 1],
        )

    return _tc_pool(batch.reshape(1, N), h, lin_W, lin_b.reshape(1, H))


# speedup vs baseline: 2.7378x; 1.0563x over previous
import functools

import jax
import jax.numpy as jnp
from jax import lax
from jax.experimental import pallas as pl
from jax.experimental.pallas import tpu as pltpu
from jax.experimental.pallas import tpu_sc as plsc

N = 10000
E = 320000
H = 128
RBF = 128
L = 3
G = 64

NC = 2
NS = 16
NW = NC * NS

CHI = 80
KJ = 8
CH = CHI * KJ

BN = 1000
BE = 2000
E2 = E // 2
NEB2 = E2 // BE

F32 = jnp.float32
HI = lax.Precision.HIGHEST

def _mesh():
    return plsc.VectorSubcoreMesh(core_axis_name="c", subcore_axis_name="s")


def _sc_gather(table, idx2d):
    M = idx2d.shape[0] * CHI
    D = table.shape[1]
    dt = table.dtype
    n_ch = M // CH

    @functools.partial(
        pl.kernel,
        mesh=_mesh(),
        out_type=jax.ShapeDtypeStruct((M, D), dt),
        scratch_types=[
            pltpu.VMEM((KJ, CHI), jnp.int32),
            pltpu.VMEM((CH, D), dt),
            pltpu.SemaphoreType.DMA,
        ],
    )
    def k(table_hbm, idx_hbm, out_hbm, idx_v, rows_v, sem):
        wid = lax.axis_index("s") * NC + lax.axis_index("c")

        @pl.loop(wid, n_ch, step=NW)
        def _(cc):
            pltpu.sync_copy(idx_hbm.at[pl.ds(cc * KJ, KJ)], idx_v)
            copies = [
                pltpu.async_copy(
                    table_hbm.at[idx_v.at[j]],
                    rows_v.at[pl.ds(j * CHI, CHI)],
                    sem,
                )
                for j in range(KJ)
            ]
            for c in copies:
                c.wait()
            pltpu.sync_copy(rows_v, out_hbm.at[pl.ds(cc * CH, CH)])

    return k(table, idx2d)


NHALF = N // NC
ACC_R = NHALF + 8


def _sc_scatter_add(m0, m1, ridx0, ridx1, zeros_acc):
    n_ch = E2 // CH
    ZB = 8

    @functools.partial(
        pl.kernel,
        mesh=_mesh(),
        out_type=jax.ShapeDtypeStruct((NC, NHALF, H), F32),
        scratch_types=[
            pltpu.VMEM((KJ, CHI), jnp.int32),
            pltpu.VMEM((CH, H), F32),
            pltpu.VMEM_SHARED((ACC_R, H), F32),
            pltpu.SemaphoreType.DMA,
        ],
    )
    def k(m0_hbm, m1_hbm, idx0_hbm, idx1_hbm, zero_hbm, out_hbm,
          idx_v, m_v, acc, sem):
        cid = lax.axis_index("c")
        sid = lax.axis_index("s")

        @pl.loop(sid, ACC_R // ZB, step=NS)
        def _(z):
            pltpu.sync_copy(
                zero_hbm.at[pl.ds(z * ZB, ZB)], acc.at[pl.ds(z * ZB, ZB)]
            )

        plsc.subcore_barrier()

        for m_hbm, idx_hbm in ((m0_hbm, idx0_hbm), (m1_hbm, idx1_hbm)):
            @pl.loop(sid, n_ch, step=NS)
            def _(cc, m_hbm=m_hbm, idx_hbm=idx_hbm):
                pltpu.sync_copy(idx_hbm.at[cid, pl.ds(cc * KJ, KJ)], idx_v)
                pltpu.sync_copy(m_hbm.at[pl.ds(cc * CH, CH)], m_v)
                for j in range(KJ):
                    pltpu.sync_copy(
                        m_v.at[pl.ds(j * CHI, CHI)],
                        acc.at[idx_v.at[j]],
                        add=True,
                    )

        plsc.subcore_barrier()

        DB = 40
        @pl.loop(sid, NHALF // DB, step=NS)
        def _(z):
            pltpu.sync_copy(
                acc.at[pl.ds(z * DB, DB)],
                out_hbm.at[cid, pl.ds(z * DB, DB)],
            )

    return k(m0, m1, ridx0, ridx1, zeros_acc)


def _silu(v):
    return v * jax.nn.sigmoid(v)


BF16 = jnp.bfloat16


def _tc_init(x3, embp, wa, wb):
    def body(x_ref, emb_ref, wa_ref, wb_ref, h0_ref, t_ref):
        xv = x_ref[0, 0, :]
        io = lax.broadcasted_iota(jnp.int32, (BN, 128), 1)
        oh = (xv[:, None] == io).astype(F32)
        h0 = jnp.dot(oh, emb_ref[...], precision=HI)
        h0_ref[...] = h0
        t_ref[0] = jnp.dot(h0, wa_ref[...], precision=HI)
        t_ref[1] = jnp.dot(h0, wb_ref[...], precision=HI)

    return pl.pallas_call(
        body,
        grid=(N // BN,),
        in_specs=[
            pl.BlockSpec((1, 1, BN), lambda i: (i, 0, 0)),
            pl.BlockSpec((128, H), lambda i: (0, 0)),
            pl.BlockSpec((H, H), lambda i: (0, 0)),
            pl.BlockSpec((H, H), lambda i: (0, 0)),
        ],
        out_specs=[
            pl.BlockSpec((BN, H), lambda i: (i, 0)),
            pl.BlockSpec((2, BN, H), lambda i: (0, i, 0)),
        ],
        out_shape=[
            jax.ShapeDtypeStruct((N, H), F32),
            jax.ShapeDtypeStruct((2, N, H), F32),
        ],
    )(x3, embp, wa, wb)


def _tc_edge(gh, edge_attr, half, wc, e2w, b1, b2):
    def body(ga_ref, gb_ref, ea_ref, wc_ref, e2_ref, b1_ref, b2_ref, m_ref):
        t = (
            ga_ref[...].astype(F32)
            + gb_ref[...].astype(F32)
            + jnp.dot(ea_ref[...], wc_ref[...],
                      preferred_element_type=F32)
            + b1_ref[...]
        )
        t = _silu(t)
        m = (
            jnp.dot(t.astype(jnp.bfloat16), e2_ref[...],
                    preferred_element_type=F32)
            + b2_ref[...]
        )
        m_ref[...] = _silu(m)

    return pl.pallas_call(
        body,
        grid=(NEB2,),
        in_specs=[
            pl.BlockSpec((BE, H), lambda i: (i, 0)),
            pl.BlockSpec((BE, H), lambda i: (i + NEB2, 0)),
            pl.BlockSpec((BE, H), lambda i, h=half: (i + h * NEB2, 0)),
            pl.BlockSpec((H, H), lambda i: (0, 0)),
            pl.BlockSpec((H, H), lambda i: (0, 0)),
            pl.BlockSpec((1, H), lambda i: (0, 0)),
            pl.BlockSpec((1, H), lambda i: (0, 0)),
        ],
        out_specs=pl.BlockSpec((BE, H), lambda i: (i, 0)),
        out_shape=jax.ShapeDtypeStruct((E2, H), F32),
    )(gh, gh, edge_attr, wc.astype(jnp.bfloat16),
      e2w.astype(jnp.bfloat16), b1, b2)


def _tc_node(h, agg_nh, n1a, n1b, nb1, n2w, nb2, wa=None, wb=None):
    with_t = wa is not None

    def body(h_ref, agg_ref, a1, a2, b1r, w2, b2r, *rest):
        if with_t:
            wa_ref, wb_ref, hn_ref, t_ref = rest
        else:
            (hn_ref,) = rest
        h_v = h_ref[...]
        agg = agg_ref[...]
        u = (
            jnp.dot(h_v, a1[...], precision=HI)
            + jnp.dot(agg, a2[...], precision=HI)
            + b1r[...]
        )
        u = _silu(u)
        hn = h_v + jnp.dot(u, w2[...], precision=HI) + b2r[...]
        hn_ref[...] = hn
        if with_t:
            t_ref[0] = jnp.dot(hn, wa_ref[...], precision=HI)
            t_ref[1] = jnp.dot(hn, wb_ref[...], precision=HI)

    in_specs = [
        pl.BlockSpec((BN, H), lambda i: (i, 0)),
        pl.BlockSpec((BN, H), lambda i: (i, 0)),
        pl.BlockSpec((H, H), lambda i: (0, 0)),
        pl.BlockSpec((H, H), lambda i: (0, 0)),
        pl.BlockSpec((1, H), lambda i: (0, 0)),
        pl.BlockSpec((H, H), lambda i: (0, 0)),
        pl.BlockSpec((1, H), lambda i: (0, 0)),
    ]
    args = [h, agg_nh, n1a, n1b, nb1, n2w, nb2]
    out_specs = [pl.BlockSpec((BN, H), lambda i: (i, 0))]
    out_shape = [jax.ShapeDtypeStruct((N, H), F32)]
    if with_t:
        in_specs += [
            pl.BlockSpec((H, H), lambda i: (0, 0)),
            pl.BlockSpec((H, H), lambda i: (0, 0)),
        ]
        args += [wa, wb]
        out_specs.append(pl.BlockSpec((2, BN, H), lambda i: (0, i, 0)))
        out_shape.append(jax.ShapeDtypeStruct((2, N, H), F32))

    res = pl.pallas_call(
        body,
        grid=(N // BN,),
        in_specs=in_specs,
        out_specs=out_specs,
        out_shape=out_shape,
    )(*args)
    return res if with_t else (res[0], None)


def _tc_pool(batch2, h, linw, linb):
    def body(b_ref, h_ref, w_ref, bias_ref, o_ref):
        bv = b_ref[0, :]
        io = lax.broadcasted_iota(jnp.int32, (N, G), 1)
        oh = (bv[:, None] == io).astype(F32)
        h_v = h_ref[...]
        sums = lax.dot_general(
            oh, h_v, (((0,), (0,)), ((), ())), precision=HI
        )
        ones = jnp.ones((N, 1), F32)
        counts = lax.dot_general(
            oh, ones, (((0,), (0,)), ((), ())), precision=HI
        )
        pooled = sums / jnp.maximum(counts, 1.0)
        o_ref[...] = (
            jnp.dot(jnp.maximum(pooled, 0.0), w_ref[...], precision=HI)
            + bias_ref[...]
        )

    return pl.pallas_call(
        body,
        grid=(1,),
        in_specs=[
            pl.BlockSpec((1, N), lambda i: (0, 0)),
            pl.BlockSpec((N, H), lambda i: (0, 0)),
            pl.BlockSpec((H, H), lambda i: (0, 0)),
            pl.BlockSpec((1, H), lambda i: (0, 0)),
        ],
        out_specs=pl.BlockSpec((G, H), lambda i: (0, 0)),
        out_shape=jax.ShapeDtypeStruct((G, H), F32),
    )(batch2, h, linw, linb)


def kernel(x, edge_index, edge_attr, batch, emb, e1_W, e1_b, e2_W, e2_b,
           n1_W, n1_b, n2_W, n2_b, lin_W, lin_b):
    x = x.astype(jnp.int32)
    row = edge_index[0]
    col = edge_index[1]
    idx_h = []
    ridx_h = []
    for h in range(2):
        rh = row[h * E2:(h + 1) * E2]
        ch = col[h * E2:(h + 1) * E2]
        idx_h.append(
            jnp.concatenate([rh, ch + N]).reshape(2 * E2 // CHI, CHI)
        )
        dummy = NHALF + (rh & 7)
        ridx_h.append(jnp.stack([
            jnp.where(rh < NHALF, rh, dummy),
            jnp.where(rh >= NHALF, rh - NHALF, dummy),
        ]).reshape(NC, E2 // CHI, CHI))
    zeros_acc = jnp.zeros((ACC_R, H), F32)
    ea16 = edge_attr.astype(jnp.bfloat16)
    embp = jnp.pad(emb, ((0, 128 - emb.shape[0]), (0, 0)))

    wa = [e1_W[l, 0:H] for l in range(L)]
    wb = [e1_W[l, H:2 * H] for l in range(L)]
    wc = [e1_W[l, 2 * H:] for l in range(L)]
    n1a = [n1_W[l, 0:H] for l in range(L)]
    n1b = [n1_W[l, H:] for l in range(L)]

    h, t = _tc_init(x.reshape(N // BN, 1, BN), embp, wa[0], wb[0])
    for l in range(L):
        tflat = t.reshape(2 * N, H)
        mh = []
        for hf in range(2):
            g = _sc_gather(tflat, idx_h[hf])
            mh.append(_tc_edge(
                g, ea16, hf, wc[l], e2_W[l],
                e1_b[l].reshape(1, H), e2_b[l].reshape(1, H),
            ))
        parts = _sc_scatter_add(mh[0], mh[1], ridx_h[0], ridx_h[1],
                                zeros_acc)
        last = l == L - 1
        h, t = _tc_node(
            h, parts.reshape(N, H), n1a[l], n1b[l], n1_b[l].reshape(1, H),
            n2_W[l], n2_b[l].reshape(1, H),
            None if last else wa[l + 1], None if last else wb[l + 1],
        )

    return _tc_pool(batch.reshape(1, N), h, lin_W, lin_b.reshape(1, H))


# speedup vs baseline: 2.9513x; 1.0780x over previous
import functools

import jax
import jax.numpy as jnp
from jax import lax
from jax.experimental import pallas as pl
from jax.experimental.pallas import tpu as pltpu
from jax.experimental.pallas import tpu_sc as plsc

N = 10000
E = 320000
H = 128
RBF = 128
L = 3
G = 64

NC = 2
NS = 16
NW = NC * NS

CHI = 80
KJ = 8
CH = CHI * KJ

BN = 1000
BE = 2000
E2 = E // 2
NEB2 = E2 // BE

F32 = jnp.float32
HI = lax.Precision.HIGHEST

def _mesh():
    return plsc.VectorSubcoreMesh(core_axis_name="c", subcore_axis_name="s")


def _sc_gather(table, idx2d):
    M = idx2d.shape[0] * CHI
    D = table.shape[1]
    dt = table.dtype
    n_ch = M // CH

    @functools.partial(
        pl.kernel,
        mesh=_mesh(),
        out_type=jax.ShapeDtypeStruct((M, D), dt),
        scratch_types=[
            pltpu.VMEM((KJ, CHI), jnp.int32),
            pltpu.VMEM((CH, D), dt),
            pltpu.SemaphoreType.DMA,
        ],
    )
    def k(table_hbm, idx_hbm, out_hbm, idx_v, rows_v, sem):
        wid = lax.axis_index("s") * NC + lax.axis_index("c")

        @pl.loop(wid, n_ch, step=NW)
        def _(cc):
            pltpu.sync_copy(idx_hbm.at[pl.ds(cc * KJ, KJ)], idx_v)
            copies = [
                pltpu.async_copy(
                    table_hbm.at[idx_v.at[j]],
                    rows_v.at[pl.ds(j * CHI, CHI)],
                    sem,
                )
                for j in range(KJ)
            ]
            for c in copies:
                c.wait()
            pltpu.sync_copy(rows_v, out_hbm.at[pl.ds(cc * CH, CH)])

    return k(table, idx2d)


NHALF = N // NC
ACC_R = NHALF + 8


def _sc_scatter_add(m, ridx3d, zeros_acc):
    n_ch = E2 // CH
    ZB = 8

    @functools.partial(
        pl.kernel,
        mesh=_mesh(),
        out_type=jax.ShapeDtypeStruct((NC, NHALF, H), F32),
        scratch_types=[
            pltpu.VMEM((KJ, CHI), jnp.int32),
            pltpu.VMEM((CH, H), F32),
            pltpu.VMEM_SHARED((ACC_R, H), F32),
            pltpu.SemaphoreType.DMA,
        ],
    )
    def k(m_hbm, idx_hbm, zero_hbm, out_hbm, idx_v, m_v, acc, sem):
        cid = lax.axis_index("c")
        sid = lax.axis_index("s")

        @pl.loop(sid, ACC_R // ZB, step=NS)
        def _(z):
            pltpu.sync_copy(
                zero_hbm.at[pl.ds(z * ZB, ZB)], acc.at[pl.ds(z * ZB, ZB)]
            )

        plsc.subcore_barrier()

        @pl.loop(sid, n_ch, step=NS)
        def _(cc):
            pltpu.sync_copy(idx_hbm.at[cid, pl.ds(cc * KJ, KJ)], idx_v)
            pltpu.sync_copy(m_hbm.at[pl.ds(cc * CH, CH)], m_v)
            for j in range(KJ):
                pltpu.sync_copy(
                    m_v.at[pl.ds(j * CHI, CHI)],
                    acc.at[idx_v.at[j]],
                    add=True,
                )

        plsc.subcore_barrier()

        DB = 40
        @pl.loop(sid, NHALF // DB, step=NS)
        def _(z):
            pltpu.sync_copy(
                acc.at[pl.ds(z * DB, DB)],
                out_hbm.at[cid, pl.ds(z * DB, DB)],
            )

    return k(m, ridx3d, zeros_acc)


def _silu(v):
    return v * jax.nn.sigmoid(v)


BF16 = jnp.bfloat16


def _tc_init(x3, embp, wa, wb):
    def body(x_ref, emb_ref, wa_ref, wb_ref, h0_ref, t_ref):
        xv = x_ref[0, 0, :]
        io = lax.broadcasted_iota(jnp.int32, (BN, 128), 1)
        oh = (xv[:, None] == io).astype(F32)
        h0 = jnp.dot(oh, emb_ref[...], precision=HI)
        h0_ref[...] = h0
        t_ref[0] = jnp.dot(h0, wa_ref[...], precision=HI)
        t_ref[1] = jnp.dot(h0, wb_ref[...], precision=HI)

    return pl.pallas_call(
        body,
        grid=(N // BN,),
        in_specs=[
            pl.BlockSpec((1, 1, BN), lambda i: (i, 0, 0)),
            pl.BlockSpec((128, H), lambda i: (0, 0)),
            pl.BlockSpec((H, H), lambda i: (0, 0)),
            pl.BlockSpec((H, H), lambda i: (0, 0)),
        ],
        out_specs=[
            pl.BlockSpec((BN, H), lambda i: (i, 0)),
            pl.BlockSpec((2, BN, H), lambda i: (0, i, 0)),
        ],
        out_shape=[
            jax.ShapeDtypeStruct((N, H), F32),
            jax.ShapeDtypeStruct((2, N, H), F32),
        ],
    )(x3, embp, wa, wb)


def _tc_edge(gh, edge_attr, half, wc, e2w, b1, b2):
    def body(ga_ref, gb_ref, ea_ref, wc_ref, e2_ref, b1_ref, b2_ref, m_ref):
        t = (
            ga_ref[...].astype(F32)
            + gb_ref[...].astype(F32)
            + jnp.dot(ea_ref[...], wc_ref[...],
                      preferred_element_type=F32)
            + b1_ref[...]
        )
        t = _silu(t)
        m = (
            jnp.dot(t.astype(jnp.bfloat16), e2_ref[...],
                    preferred_element_type=F32)
            + b2_ref[...]
        )
        m_ref[...] = _silu(m)

    return pl.pallas_call(
        body,
        grid=(NEB2,),
        in_specs=[
            pl.BlockSpec((BE, H), lambda i: (i, 0)),
            pl.BlockSpec((BE, H), lambda i: (i + NEB2, 0)),
            pl.BlockSpec((BE, H), lambda i, h=half: (i + h * NEB2, 0)),
            pl.BlockSpec((H, H), lambda i: (0, 0)),
            pl.BlockSpec((H, H), lambda i: (0, 0)),
            pl.BlockSpec((1, H), lambda i: (0, 0)),
            pl.BlockSpec((1, H), lambda i: (0, 0)),
        ],
        out_specs=pl.BlockSpec((BE, H), lambda i: (i, 0)),
        out_shape=jax.ShapeDtypeStruct((E2, H), F32),
    )(gh, gh, edge_attr, wc.astype(jnp.bfloat16),
      e2w.astype(jnp.bfloat16), b1, b2)


def _tc_node(h, agg_a, agg_b, n1a, n1b, nb1, n2w, nb2, wa=None, wb=None):
    with_t = wa is not None

    def body(h_ref, aa_ref, ab_ref, a1, a2, b1r, w2, b2r, *rest):
        if with_t:
            wa_ref, wb_ref, hn_ref, t_ref = rest
        else:
            (hn_ref,) = rest
        h_v = h_ref[...]
        agg = aa_ref[...] + ab_ref[...]
        u = (
            jnp.dot(h_v, a1[...], precision=HI)
            + jnp.dot(agg, a2[...], precision=HI)
            + b1r[...]
        )
        u = _silu(u)
        hn = h_v + jnp.dot(u, w2[...], precision=HI) + b2r[...]
        hn_ref[...] = hn
        if with_t:
            t_ref[0] = jnp.dot(hn, wa_ref[...], precision=HI)
            t_ref[1] = jnp.dot(hn, wb_ref[...], precision=HI)

    in_specs = [
        pl.BlockSpec((BN, H), lambda i: (i, 0)),
        pl.BlockSpec((BN, H), lambda i: (i, 0)),
        pl.BlockSpec((BN, H), lambda i: (i, 0)),
        pl.BlockSpec((H, H), lambda i: (0, 0)),
        pl.BlockSpec((H, H), lambda i: (0, 0)),
        pl.BlockSpec((1, H), lambda i: (0, 0)),
        pl.BlockSpec((H, H), lambda i: (0, 0)),
        pl.BlockSpec((1, H), lambda i: (0, 0)),
    ]
    args = [h, agg_a, agg_b, n1a, n1b, nb1, n2w, nb2]
    out_specs = [pl.BlockSpec((BN, H), lambda i: (i, 0))]
    out_shape = [jax.ShapeDtypeStruct((N, H), F32)]
    if with_t:
        in_specs += [
            pl.BlockSpec((H, H), lambda i: (0, 0)),
            pl.BlockSpec((H, H), lambda i: (0, 0)),
        ]
        args += [wa, wb]
        out_specs.append(pl.BlockSpec((2, BN, H), lambda i: (0, i, 0)))
        out_shape.append(jax.ShapeDtypeStruct((2, N, H), F32))

    res = pl.pallas_call(
        body,
        grid=(N // BN,),
        in_specs=in_specs,
        out_specs=out_specs,
        out_shape=out_shape,
    )(*args)
    return res if with_t else (res[0], None)


def _tc_pool(batch2, h, linw, linb):
    def body(b_ref, h_ref, w_ref, bias_ref, o_ref):
        bv = b_ref[0, :]
        io = lax.broadcasted_iota(jnp.int32, (N, G), 1)
        oh = (bv[:, None] == io).astype(F32)
        h_v = h_ref[...]
        sums = lax.dot_general(
            oh, h_v, (((0,), (0,)), ((), ())), precision=HI
        )
        ones = jnp.ones((N, 1), F32)
        counts = lax.dot_general(
            oh, ones, (((0,), (0,)), ((), ())), precision=HI
        )
        pooled = sums / jnp.maximum(counts, 1.0)
        o_ref[...] = (
            jnp.dot(jnp.maximum(pooled, 0.0), w_ref[...], precision=HI)
            + bias_ref[...]
        )

    return pl.pallas_call(
        body,
        grid=(1,),
        in_specs=[
            pl.BlockSpec((1, N), lambda i: (0, 0)),
            pl.BlockSpec((N, H), lambda i: (0, 0)),
            pl.BlockSpec((H, H), lambda i: (0, 0)),
            pl.BlockSpec((1, H), lambda i: (0, 0)),
        ],
        out_specs=pl.BlockSpec((G, H), lambda i: (0, 0)),
        out_shape=jax.ShapeDtypeStruct((G, H), F32),
    )(batch2, h, linw, linb)


def kernel(x, edge_index, edge_attr, batch, emb, e1_W, e1_b, e2_W, e2_b,
           n1_W, n1_b, n2_W, n2_b, lin_W, lin_b):
    x = x.astype(jnp.int32)
    row = edge_index[0]
    col = edge_index[1]
    idx_h = []
    ridx_h = []
    for h in range(2):
        rh = row[h * E2:(h + 1) * E2]
        ch = col[h * E2:(h + 1) * E2]
        idx_h.append(
            jnp.concatenate([rh, ch + N]).reshape(2 * E2 // CHI, CHI)
        )
        dummy = NHALF + (rh & 7)
        ridx_h.append(jnp.stack([
            jnp.where(rh < NHALF, rh, dummy),
            jnp.where(rh >= NHALF, rh - NHALF, dummy),
        ]).reshape(NC, E2 // CHI, CHI))
    zeros_acc = jnp.zeros((ACC_R, H), F32)
    ea16 = edge_attr.astype(jnp.bfloat16)
    embp = jnp.pad(emb, ((0, 128 - emb.shape[0]), (0, 0)))

    wa = [e1_W[l, 0:H] for l in range(L)]
    wb = [e1_W[l, H:2 * H] for l in range(L)]
    wc = [e1_W[l, 2 * H:] for l in range(L)]
    n1a = [n1_W[l, 0:H] for l in range(L)]
    n1b = [n1_W[l, H:] for l in range(L)]

    h, t = _tc_init(x.reshape(N // BN, 1, BN), embp, wa[0], wb[0])
    for l in range(L):
        tflat = t.reshape(2 * N, H)
        ph = []
        for hf in range(2):
            g = _sc_gather(tflat, idx_h[hf])
            m = _tc_edge(
                g, ea16, hf, wc[l], e2_W[l],
                e1_b[l].reshape(1, H), e2_b[l].reshape(1, H),
            )
            ph.append(_sc_scatter_add(m, ridx_h[hf], zeros_acc))
        last = l == L - 1
        h, t = _tc_node(
            h, ph[0].reshape(N, H), ph[1].reshape(N, H),
            n1a[l], n1b[l], n1_b[l].reshape(1, H),
            n2_W[l], n2_b[l].reshape(1, H),
            None if last else wa[l + 1], None if last else wb[l + 1],
        )

    return _tc_pool(batch.reshape(1, N), h, lin_W, lin_b.reshape(1, H))


# speedup vs baseline: 3.0710x; 1.0406x over previous
import functools

import jax
import jax.numpy as jnp
from jax import lax
from jax.experimental import pallas as pl
from jax.experimental.pallas import tpu as pltpu
from jax.experimental.pallas import tpu_sc as plsc

N = 10000
E = 320000
H = 128
RBF = 128
L = 3
G = 64

NC = 2
NS = 16
NW = NC * NS

CHI = 80
KJ = 8
CH = CHI * KJ

BN = 1000
BE = 2000
E2 = E // 2
NEB2 = E2 // BE

F32 = jnp.float32
HI = lax.Precision.HIGHEST

def _mesh():
    return plsc.VectorSubcoreMesh(core_axis_name="c", subcore_axis_name="s")


def _sc_gather(table, idx2d):
    M = idx2d.shape[0] * CHI
    D = table.shape[1]
    dt = table.dtype
    n_ch = M // CH

    @functools.partial(
        pl.kernel,
        mesh=_mesh(),
        out_type=jax.ShapeDtypeStruct((M, D), dt),
        scratch_types=[
            pltpu.VMEM((KJ, CHI), jnp.int32),
            pltpu.VMEM((CH, D), dt),
            pltpu.SemaphoreType.DMA,
        ],
    )
    def k(table_hbm, idx_hbm, out_hbm, idx_v, rows_v, sem):
        wid = lax.axis_index("s") * NC + lax.axis_index("c")

        @pl.loop(wid, n_ch, step=NW)
        def _(cc):
            pltpu.sync_copy(idx_hbm.at[pl.ds(cc * KJ, KJ)], idx_v)
            copies = [
                pltpu.async_copy(
                    table_hbm.at[idx_v.at[j]],
                    rows_v.at[pl.ds(j * CHI, CHI)],
                    sem,
                )
                for j in range(KJ)
            ]
            for c in copies:
                c.wait()
            pltpu.sync_copy(rows_v, out_hbm.at[pl.ds(cc * CH, CH)])

    return k(table, idx2d)


NHALF = N // NC
ACC_R = NHALF + 8


def _sc_scatter_add(m, ridx3d, zeros_acc):
    n_ch = E2 // CH
    ZB = 8

    @functools.partial(
        pl.kernel,
        mesh=_mesh(),
        out_type=jax.ShapeDtypeStruct((NC, NHALF, H), F32),
        scratch_types=[
            pltpu.VMEM((KJ, CHI), jnp.int32),
            pltpu.VMEM((CH, H), F32),
            pltpu.VMEM_SHARED((ACC_R, H), F32),
            pltpu.SemaphoreType.DMA,
            pltpu.SemaphoreType.DMA,
        ],
    )
    def k(m_hbm, idx_hbm, zero_hbm, out_hbm, idx_v, m_v, acc, sem, sem2):
        cid = lax.axis_index("c")
        sid = lax.axis_index("s")

        @pl.loop(sid, ACC_R // ZB, step=NS)
        def _(z):
            pltpu.sync_copy(
                zero_hbm.at[pl.ds(z * ZB, ZB)], acc.at[pl.ds(z * ZB, ZB)]
            )

        plsc.subcore_barrier()

        @pl.loop(sid, n_ch, step=NS)
        def _(cc):
            c_i = pltpu.async_copy(
                idx_hbm.at[cid, pl.ds(cc * KJ, KJ)], idx_v, sem
            )
            c_m = pltpu.async_copy(m_hbm.at[pl.ds(cc * CH, CH)], m_v, sem2)
            c_i.wait()
            c_m.wait()
            adds = [
                pltpu.async_copy(
                    m_v.at[pl.ds(j * CHI, CHI)],
                    acc.at[idx_v.at[j]],
                    sem,
                    add=True,
                )
                for j in range(KJ)
            ]
            for a in adds:
                a.wait()

        plsc.subcore_barrier()

        DB = 40
        @pl.loop(sid, NHALF // DB, step=NS)
        def _(z):
            pltpu.sync_copy(
                acc.at[pl.ds(z * DB, DB)],
                out_hbm.at[cid, pl.ds(z * DB, DB)],
            )

    return k(m, ridx3d, zeros_acc)


def _silu(v):
    return v * jax.nn.sigmoid(v)


BF16 = jnp.bfloat16


def _tc_init(x3, embp, wa, wb):
    def body(x_ref, emb_ref, wa_ref, wb_ref, h0_ref, t_ref):
        xv = x_ref[0, 0, :]
        io = lax.broadcasted_iota(jnp.int32, (BN, 128), 1)
        oh = (xv[:, None] == io).astype(F32)
        h0 = jnp.dot(oh, emb_ref[...], precision=HI)
        h0_ref[...] = h0
        t_ref[0] = jnp.dot(h0, wa_ref[...], precision=HI)
        t_ref[1] = jnp.dot(h0, wb_ref[...], precision=HI)

    return pl.pallas_call(
        body,
        grid=(N // BN,),
        in_specs=[
            pl.BlockSpec((1, 1, BN), lambda i: (i, 0, 0)),
            pl.BlockSpec((128, H), lambda i: (0, 0)),
            pl.BlockSpec((H, H), lambda i: (0, 0)),
            pl.BlockSpec((H, H), lambda i: (0, 0)),
        ],
        out_specs=[
            pl.BlockSpec((BN, H), lambda i: (i, 0)),
            pl.BlockSpec((2, BN, H), lambda i: (0, i, 0)),
        ],
        out_shape=[
            jax.ShapeDtypeStruct((N, H), F32),
            jax.ShapeDtypeStruct((2, N, H), F32),
        ],
    )(x3, embp, wa, wb)


def _tc_edge(gh, edge_attr, half, wc, e2w, b1, b2):
    def body(ga_ref, gb_ref, ea_ref, wc_ref, e2_ref, b1_ref, b2_ref, m_ref):
        t = (
            ga_ref[...].astype(F32)
            + gb_ref[...].astype(F32)
            + jnp.dot(ea_ref[...], wc_ref[...],
                      preferred_element_type=F32)
            + b1_ref[...]
        )
        t = _silu(t)
        m = (
            jnp.dot(t.astype(jnp.bfloat16), e2_ref[...],
                    preferred_element_type=F32)
            + b2_ref[...]
        )
        m_ref[...] = _silu(m)

    return pl.pallas_call(
        body,
        grid=(NEB2,),
        in_specs=[
            pl.BlockSpec((BE, H), lambda i: (i, 0)),
            pl.BlockSpec((BE, H), lambda i: (i + NEB2, 0)),
            pl.BlockSpec((BE, H), lambda i, h=half: (i + h * NEB2, 0)),
            pl.BlockSpec((H, H), lambda i: (0, 0)),
            pl.BlockSpec((H, H), lambda i: (0, 0)),
            pl.BlockSpec((1, H), lambda i: (0, 0)),
            pl.BlockSpec((1, H), lambda i: (0, 0)),
        ],
        out_specs=pl.BlockSpec((BE, H), lambda i: (i, 0)),
        out_shape=jax.ShapeDtypeStruct((E2, H), F32),
    )(gh, gh, edge_attr, wc.astype(jnp.bfloat16),
      e2w.astype(jnp.bfloat16), b1, b2)


def _tc_node(h, agg_a, agg_b, n1a, n1b, nb1, n2w, nb2, wa=None, wb=None):
    with_t = wa is not None

    def body(h_ref, aa_ref, ab_ref, a1, a2, b1r, w2, b2r, *rest):
        if with_t:
            wa_ref, wb_ref, hn_ref, t_ref = rest
        else:
            (hn_ref,) = rest
        h_v = h_ref[...]
        agg = aa_ref[...] + ab_ref[...]
        u = (
            jnp.dot(h_v, a1[...], precision=HI)
            + jnp.dot(agg, a2[...], precision=HI)
            + b1r[...]
        )
        u = _silu(u)
        hn = h_v + jnp.dot(u, w2[...], precision=HI) + b2r[...]
        hn_ref[...] = hn
        if with_t:
            t_ref[0] = jnp.dot(hn, wa_ref[...], precision=HI)
            t_ref[1] = jnp.dot(hn, wb_ref[...], precision=HI)

    in_specs = [
        pl.BlockSpec((BN, H), lambda i: (i, 0)),
        pl.BlockSpec((BN, H), lambda i: (i, 0)),
        pl.BlockSpec((BN, H), lambda i: (i, 0)),
        pl.BlockSpec((H, H), lambda i: (0, 0)),
        pl.BlockSpec((H, H), lambda i: (0, 0)),
        pl.BlockSpec((1, H), lambda i: (0, 0)),
        pl.BlockSpec((H, H), lambda i: (0, 0)),
        pl.BlockSpec((1, H), lambda i: (0, 0)),
    ]
    args = [h, agg_a, agg_b, n1a, n1b, nb1, n2w, nb2]
    out_specs = [pl.BlockSpec((BN, H), lambda i: (i, 0))]
    out_shape = [jax.ShapeDtypeStruct((N, H), F32)]
    if with_t:
        in_specs += [
            pl.BlockSpec((H, H), lambda i: (0, 0)),
            pl.BlockSpec((H, H), lambda i: (0, 0)),
        ]
        args += [wa, wb]
        out_specs.append(pl.BlockSpec((2, BN, H), lambda i: (0, i, 0)))
        out_shape.append(jax.ShapeDtypeStruct((2, N, H), F32))

    res = pl.pallas_call(
        body,
        grid=(N // BN,),
        in_specs=in_specs,
        out_specs=out_specs,
        out_shape=out_shape,
    )(*args)
    return res if with_t else (res[0], None)


def _tc_pool(batch2, h, linw, linb):
    def body(b_ref, h_ref, w_ref, bias_ref, o_ref):
        bv = b_ref[0, :]
        io = lax.broadcasted_iota(jnp.int32, (N, G), 1)
        oh = (bv[:, None] == io).astype(F32)
        h_v = h_ref[...]
        sums = lax.dot_general(
            oh, h_v, (((0,), (0,)), ((), ())), precision=HI
        )
        ones = jnp.ones((N, 1), F32)
        counts = lax.dot_general(
            oh, ones, (((0,), (0,)), ((), ())), precision=HI
        )
        pooled = sums / jnp.maximum(counts, 1.0)
        o_ref[...] = (
            jnp.dot(jnp.maximum(pooled, 0.0), w_ref[...], precision=HI)
            + bias_ref[...]
        )

    return pl.pallas_call(
        body,
        grid=(1,),
        in_specs=[
            pl.BlockSpec((1, N), lambda i: (0, 0)),
            pl.BlockSpec((N, H), lambda i: (0, 0)),
            pl.BlockSpec((H, H), lambda i: (0, 0)),
            pl.BlockSpec((1, H), lambda i: (0, 0)),
        ],
        out_specs=pl.BlockSpec((G, H), lambda i: (0, 0)),
        out_shape=jax.ShapeDtypeStruct((G, H), F32),
    )(batch2, h, linw, linb)


def kernel(x, edge_index, edge_attr, batch, emb, e1_W, e1_b, e2_W, e2_b,
           n1_W, n1_b, n2_W, n2_b, lin_W, lin_b):
    x = x.astype(jnp.int32)
    row = edge_index[0]
    col = edge_index[1]
    idx_h = []
    ridx_h = []
    for h in range(2):
        rh = row[h * E2:(h + 1) * E2]
        ch = col[h * E2:(h + 1) * E2]
        idx_h.append(
            jnp.concatenate([rh, ch + N]).reshape(2 * E2 // CHI, CHI)
        )
        dummy = NHALF + (rh & 7)
        ridx_h.append(jnp.stack([
            jnp.where(rh < NHALF, rh, dummy),
            jnp.where(rh >= NHALF, rh - NHALF, dummy),
        ]).reshape(NC, E2 // CHI, CHI))
    zeros_acc = jnp.zeros((ACC_R, H), F32)
    ea16 = edge_attr.astype(jnp.bfloat16)
    embp = jnp.pad(emb, ((0, 128 - emb.shape[0]), (0, 0)))

    wa = [e1_W[l, 0:H] for l in range(L)]
    wb = [e1_W[l, H:2 * H] for l in range(L)]
    wc = [e1_W[l, 2 * H:] for l in range(L)]
    n1a = [n1_W[l, 0:H] for l in range(L)]
    n1b = [n1_W[l, H:] for l in range(L)]

    h, t = _tc_init(x.reshape(N // BN, 1, BN), embp, wa[0], wb[0])
    for l in range(L):
        tflat = t.reshape(2 * N, H)
        ph = []
        for hf in range(2):
            g = _sc_gather(tflat, idx_h[hf])
            m = _tc_edge(
                g, ea16, hf, wc[l], e2_W[l],
                e1_b[l].reshape(1, H), e2_b[l].reshape(1, H),
            )
            ph.append(_sc_scatter_add(m, ridx_h[hf], zeros_acc))
        last = l == L - 1
        h, t = _tc_node(
            h, ph[0].reshape(N, H), ph[1].reshape(N, H),
            n1a[l], n1b[l], n1_b[l].reshape(1, H),
            n2_W[l], n2_b[l].reshape(1, H),
            None if last else wa[l + 1], None if last else wb[l + 1],
        )

    return _tc_pool(batch.reshape(1, N), h, lin_W, lin_b.reshape(1, H))
